# speedup vs baseline: 1.2805x; 1.2805x over previous
import functools

import jax
import jax.numpy as jnp
from jax import lax
from jax.experimental import pallas as pl
from jax.experimental.pallas import tpu as pltpu
from jax.experimental.pallas import tpu_sc as plsc


def kernel(indices, table):
    B, S = indices.shape
    V, D = table.shape
    N = B * S

    info = plsc.get_sparse_core_info()
    NC, NS = info.num_cores, info.num_subcores
    NW = NC * NS
    CHUNK = 128
    assert N % (NW * CHUNK) == 0
    n_chunks = N // (NW * CHUNK)

    idx3 = indices.reshape(NW, n_chunks, CHUNK)

    mesh = plsc.VectorSubcoreMesh(core_axis_name="c", subcore_axis_name="s")

    @functools.partial(
        pl.kernel,
        mesh=mesh,
        out_type=jax.ShapeDtypeStruct((N, D), jnp.float32),
        scratch_types=[
            pltpu.VMEM((n_chunks, CHUNK), jnp.int32),
            pltpu.VMEM((CHUNK, D), jnp.float32),
            pltpu.SemaphoreType.DMA,
        ],
    )
    def sc_gather(idx_hbm, table_hbm, out_hbm, idx_v, rows_v, sem):
        wid = lax.axis_index("s") * NC + lax.axis_index("c")
        base = wid * (n_chunks * CHUNK)
        pltpu.sync_copy(idx_hbm.at[wid], idx_v)

        def body(j, carry):
            pltpu.async_copy(table_hbm.at[idx_v.at[j]], rows_v, sem).wait()
            pltpu.sync_copy(rows_v, out_hbm.at[pl.ds(base + j * CHUNK, CHUNK)])
            return carry

        lax.fori_loop(0, n_chunks, body, 0)

    out = sc_gather(idx3, table)
    return out.reshape(B, S, D)


# speedup vs baseline: 1.8724x; 1.4622x over previous
import functools

import jax
import jax.numpy as jnp
from jax import lax
from jax.experimental import pallas as pl
from jax.experimental.pallas import tpu as pltpu
from jax.experimental.pallas import tpu_sc as plsc


def kernel(indices, table):
    B, S = indices.shape
    V, D = table.shape
    N = B * S

    info = plsc.get_sparse_core_info()
    NC, NS = info.num_cores, info.num_subcores
    NW = NC * NS
    CHUNK = 128
    assert N % (NW * CHUNK * 2) == 0
    n_chunks = N // (NW * CHUNK)
    n2 = n_chunks // 2

    idx3 = indices.reshape(NW, n_chunks, CHUNK)

    mesh = plsc.VectorSubcoreMesh(core_axis_name="c", subcore_axis_name="s")

    @functools.partial(
        pl.kernel,
        mesh=mesh,
        out_type=jax.ShapeDtypeStruct((N, D), jnp.float32),
        scratch_types=[
            pltpu.VMEM((n_chunks, CHUNK), jnp.int32),
            pltpu.VMEM((CHUNK, D), jnp.float32),
            pltpu.VMEM((CHUNK, D), jnp.float32),
            pltpu.SemaphoreType.DMA,
            pltpu.SemaphoreType.DMA,
            pltpu.SemaphoreType.DMA,
            pltpu.SemaphoreType.DMA,
        ],
    )
    def sc_gather(idx_hbm, table_hbm, out_hbm, idx_v, rows0, rows1,
                  gsem0, gsem1, wsem0, wsem1):
        wid = lax.axis_index("s") * NC + lax.axis_index("c")
        base = wid * (n_chunks * CHUNK)
        pltpu.sync_copy(idx_hbm.at[wid], idx_v)

        def gather(j, buf, sem):
            return pltpu.make_async_copy(table_hbm.at[idx_v.at[j]], buf, sem)

        def write(j, buf, sem):
            return pltpu.make_async_copy(
                buf, out_hbm.at[pl.ds(base + j * CHUNK, CHUNK)], sem)

        gather(0, rows0, gsem0).start()
        gather(1, rows1, gsem1).start()
        gather(0, rows0, gsem0).wait()
        write(0, rows0, wsem0).start()
        write(0, rows0, wsem0).wait()
        gather(2, rows0, gsem0).start()
        gather(1, rows1, gsem1).wait()
        write(1, rows1, wsem1).start()

        def body(g, carry):
            j = 2 * g
            write(j - 1, rows1, wsem1).wait()
            gather(j + 1, rows1, gsem1).start()
            gather(j, rows0, gsem0).wait()
            write(j, rows0, wsem0).start()
            write(j, rows0, wsem0).wait()
            gather(j + 2, rows0, gsem0).start()
            gather(j + 1, rows1, gsem1).wait()
            write(j + 1, rows1, wsem1).start()
            return carry

        lax.fori_loop(1, n2 - 1, body, 0)

        j = n_chunks - 2
        write(j - 1, rows1, wsem1).wait()
        gather(j + 1, rows1, gsem1).start()
        gather(j, rows0, gsem0).wait()
        write(j, rows0, wsem0).start()
        write(j, rows0, wsem0).wait()
        gather(j + 1, rows1, gsem1).wait()
        write(j + 1, rows1, wsem1).start()
        write(j + 1, rows1, wsem1).wait()

    out = sc_gather(idx3, table)
    return out.reshape(B, S, D)
